# baseline (speedup 1.0000x reference)
import functools

import jax
import jax.numpy as jnp
from jax import lax
from jax.experimental import pallas as pl
from jax.experimental.pallas import tpu as pltpu
from jax.experimental.pallas import tpu_sc as plsc

_INTERPRET = False

B, T, H = 1, 2048, 1024
I = 2048
E = 8
K = 2
NT_TOK = B * T
NP = NT_TOK * K
TILE = 256
TILE_SHIFT = 8
NT_E = NP // TILE + E
PE = NT_E * TILE
NT_S = NT_TOK // TILE
NT = NT_E + NT_S
PTOT = PE + NT_TOK
EPAD = 128

NC, NS = 2, 16
NW = NC * NS
L = 16

_SC_MESH = dict(core_axis_name="c", subcore_axis_name="s",
                num_cores=NC, num_subcores=NS)


def _router_kernel(x_ref, wg_ref, e1_ref, e2_ref, v1_ref, v2_ref, aux_ref):
    x = x_ref[...]
    wg = wg_ref[...]
    logits = lax.dot_general(x, wg, (((1,), (1,)), ((), ())),
                             preferred_element_type=jnp.float32)
    col = lax.broadcasted_iota(jnp.int32, logits.shape, 1)
    valid = col < E
    z = jnp.where(valid, logits, -1e30)
    zmax = jnp.max(z, axis=1, keepdims=True)
    p = jnp.where(valid, jnp.exp(z - zmax), 0.0)
    probs = p / jnp.sum(p, axis=1, keepdims=True)
    v1 = jnp.max(probs, axis=1, keepdims=True)
    e1 = jnp.min(jnp.where(probs >= v1, col, EPAD), axis=1, keepdims=True)
    probs2 = jnp.where(col == e1, -1.0, probs)
    v2 = jnp.max(probs2, axis=1, keepdims=True)
    e2 = jnp.min(jnp.where(probs2 >= v2, col, EPAD), axis=1, keepdims=True)
    e1_ref[...] = e1
    e2_ref[...] = e2
    v1_ref[...] = v1
    v2_ref[...] = v2
    cnt = jnp.sum((col == e1).astype(jnp.float32)
                  + (col == e2).astype(jnp.float32), axis=0, keepdims=True)
    imp = jnp.mean(probs, axis=0, keepdims=True)
    aux_ref[...] = jnp.sum(imp * cnt, axis=1, keepdims=True) * (
        float(E) / float(NT_TOK * K))


def _router(x_flat, Wg):
    wg_pad = jnp.zeros((EPAD, H), jnp.float32).at[:E].set(Wg)
    kern = pl.pallas_call(
        _router_kernel,
        out_shape=(
            jax.ShapeDtypeStruct((T, 1), jnp.int32),
            jax.ShapeDtypeStruct((T, 1), jnp.int32),
            jax.ShapeDtypeStruct((T, 1), jnp.float32),
            jax.ShapeDtypeStruct((T, 1), jnp.float32),
            jax.ShapeDtypeStruct((1, 1), jnp.float32),
        ),
        interpret=_INTERPRET,
    )
    return kern(x_flat, wg_pad)


def _route_kernel(e1_hbm, e2_hbm, v1_hbm, v2_hbm,
                  tok_out, w_out, em_out, valid_out, pos1_out, pos2_out,
                  e1_v, e2_v, v1_v, v2_v, tok_v, w_v, em_v, valid_v,
                  pos1_v, pos2_v):
    cid = lax.axis_index("c")
    sid = lax.axis_index("s")

    @pl.when(jnp.logical_and(cid == 0, sid == 0))
    def _():
        pltpu.sync_copy(e1_hbm, e1_v)
        pltpu.sync_copy(e2_hbm, e2_v)
        pltpu.sync_copy(v1_hbm, v1_v)
        pltpu.sync_copy(v2_hbm, v2_v)

        zero_i = jnp.zeros((L,), jnp.int32)
        zero_f = jnp.zeros((L,), jnp.float32)
        one_f = jnp.ones((L,), jnp.float32)
        lane = lax.iota(jnp.int32, L)

        def zero_body(i, _):
            tok_v[pl.ds(i * L, L)] = zero_i
            w_v[pl.ds(i * L, L)] = zero_f
            return 0
        lax.fori_loop(0, PE // L, zero_body, 0, unroll=4)

        def shared_body(i, _):
            w_v[pl.ds(PE + i * L, L)] = one_f
            return 0
        lax.fori_loop(0, NT_TOK // L, shared_body, 0, unroll=4)

        def cnt_body(i, c):
            a = e1_v[pl.ds(i * L, L)]
            b = e2_v[pl.ds(i * L, L)]
            return tuple(
                c[e]
                + jnp.sum(jnp.where(a == e, 1, 0))
                + jnp.sum(jnp.where(b == e, 1, 0))
                for e in range(E))
        counts = lax.fori_loop(0, NT_TOK // L, cnt_body,
                               (jnp.int32(0),) * E)

        tiles = [lax.shift_right_logical(counts[e] + (TILE - 1), TILE_SHIFT)
                 for e in range(E)]
        bases = [jnp.int32(0)]
        for e in range(E - 1):
            bases.append(bases[e] + lax.shift_left(tiles[e], TILE_SHIFT))
        cum_tiles = [tiles[0]]
        for e in range(1, E):
            cum_tiles.append(cum_tiles[e - 1] + tiles[e])
        used = cum_tiles[E - 1]

        for j in range(NT // L):
            ti = j * L + lane
            em = zero_i
            for e in range(E):
                em = em + jnp.where(ti >= cum_tiles[e], 1, 0)
            is_shared = ti >= NT_E
            em_v[pl.ds(j * L, L)] = jnp.where(is_shared, E, em)
            vd = jnp.logical_or(ti < used, is_shared)
            valid_v[pl.ds(j * L, L)] = jnp.where(vd, 1, 0)

        def make_pos_body(e_ref, v_ref, p_ref):
            def body(i, b):
                ids = e_ref[pl.ds(i * L, L)]
                wv = v_ref[pl.ds(i * L, L)]
                pos = zero_i
                nb = []
                for e in range(E):
                    m = ids == e
                    ones = jnp.where(m, 1, 0)
                    pf = plsc.cumsum(ones)
                    pos = jnp.where(m, b[e] + pf - 1, pos)
                    nb.append(b[e] + jnp.sum(ones))
                tok = i * L + lane
                plsc.store_scatter(tok_v, [pos], tok)
                plsc.store_scatter(w_v, [pos], wv)
                p_ref[pl.ds(i * L, L)] = pos
                return tuple(nb)
            return body

        bases = lax.fori_loop(0, NT_TOK // L,
                              make_pos_body(e1_v, v1_v, pos1_v),
                              tuple(bases))
        lax.fori_loop(0, NT_TOK // L,
                      make_pos_body(e2_v, v2_v, pos2_v), bases)

        pltpu.sync_copy(tok_v, tok_out)
        pltpu.sync_copy(w_v, w_out)
        pltpu.sync_copy(em_v, em_out)
        pltpu.sync_copy(valid_v, valid_out)
        pltpu.sync_copy(pos1_v, pos1_out)
        pltpu.sync_copy(pos2_v, pos2_out)


def _route_sort(e1, e2, v1, v2):
    kern = functools.partial(
        pl.kernel,
        out_type=(
            jax.ShapeDtypeStruct((PE,), jnp.int32),
            jax.ShapeDtypeStruct((PTOT,), jnp.float32),
            jax.ShapeDtypeStruct((NT,), jnp.int32),
            jax.ShapeDtypeStruct((NT,), jnp.int32),
            jax.ShapeDtypeStruct((NT_TOK,), jnp.int32),
            jax.ShapeDtypeStruct((NT_TOK,), jnp.int32),
        ),
        mesh=plsc.VectorSubcoreMesh(**_SC_MESH),
        compiler_params=pltpu.CompilerParams(needs_layout_passes=False),
        scratch_types=[
            pltpu.VMEM((NT_TOK,), jnp.int32),
            pltpu.VMEM((NT_TOK,), jnp.int32),
            pltpu.VMEM((NT_TOK,), jnp.float32),
            pltpu.VMEM((NT_TOK,), jnp.float32),
            pltpu.VMEM((PE,), jnp.int32),
            pltpu.VMEM((PTOT,), jnp.float32),
            pltpu.VMEM((NT,), jnp.int32),
            pltpu.VMEM((NT,), jnp.int32),
            pltpu.VMEM((NT_TOK,), jnp.int32),
            pltpu.VMEM((NT_TOK,), jnp.int32),
        ],
        interpret=_INTERPRET,
    )(_route_kernel)
    return kern(e1, e2, v1, v2)


_GCH = 48
_GNC = PE // NW // _GCH
_SL = H // 128


def _gather_kernel(tok_hbm, x_hbm, out_hbm, idx_v,
                   rows0, rows1, rows2, rows3,
                   gs0, gs1, gs2, gs3, ws0, ws1, ws2, ws3):
    wid = lax.axis_index("s") * NC + lax.axis_index("c")
    rows_per_w = PE // NW
    rows = (rows0, rows1, rows2, rows3)
    gsem = (gs0, gs1, gs2, gs3)
    wsem = (ws0, ws1, ws2, ws3)

    pltpu.sync_copy(tok_hbm.at[pl.ds(wid * _GNC, _GNC)], idx_v)

    gd = [pltpu.async_copy(x_hbm.at[idx_v.at[c]], rows[c], gsem[c])
          for c in range(_GNC)]
    wd = []
    for c in range(_GNC):
        gd[c].wait()
        base = wid * rows_per_w + c * _GCH
        wd.append(pltpu.async_copy(rows[c], out_hbm.at[pl.ds(base, _GCH)],
                                   wsem[c]))
    for d in wd:
        d.wait()


def _gather_rows(tok_sorted, x_pk):
    hw = H // 2
    kern = functools.partial(
        pl.kernel,
        out_type=jax.ShapeDtypeStruct((PE, hw), jnp.int32),
        mesh=plsc.VectorSubcoreMesh(**_SC_MESH),
        compiler_params=pltpu.CompilerParams(needs_layout_passes=False),
        scratch_types=[
            pltpu.VMEM((_GNC, _GCH), jnp.int32),
            pltpu.VMEM((_GCH, hw), jnp.int32),
            pltpu.VMEM((_GCH, hw), jnp.int32),
            pltpu.VMEM((_GCH, hw), jnp.int32),
            pltpu.VMEM((_GCH, hw), jnp.int32),
            pltpu.SemaphoreType.DMA,
            pltpu.SemaphoreType.DMA,
            pltpu.SemaphoreType.DMA,
            pltpu.SemaphoreType.DMA,
            pltpu.SemaphoreType.DMA,
            pltpu.SemaphoreType.DMA,
            pltpu.SemaphoreType.DMA,
            pltpu.SemaphoreType.DMA,
        ],
        interpret=_INTERPRET,
    )(_gather_kernel)
    return kern(tok_sorted.reshape(PE // _GCH, _GCH), x_pk)


def _mlp_kernel(em_ref, valid_ref, xg_ref, xs_ref, w1_ref, w2_ref, ws_ref,
                out_ref):
    i = pl.program_id(0)

    @pl.when(valid_ref[i] != 0)
    def _():
        is_expert = (i < NT_E)
        x = jnp.where(is_expert, xg_ref[...],
                      xs_ref[...].astype(jnp.bfloat16))
        h = lax.dot_general(x, w1_ref[0], (((1,), (1,)), ((), ())),
                            preferred_element_type=jnp.float32)
        a = h * jax.nn.sigmoid(h)
        o = lax.dot_general(a.astype(jnp.bfloat16), w2_ref[0],
                            (((1,), (1,)), ((), ())),
                            preferred_element_type=jnp.float32)
        out_ref[...] = o * ws_ref[0, 0][:, None]


def _expert_mlp(em, valid, xg, x_flat, W1all, W2all, wslot):
    grid_spec = pltpu.PrefetchScalarGridSpec(
        num_scalar_prefetch=2,
        grid=(NT,),
        in_specs=[
            pl.BlockSpec((TILE, H),
                         lambda i, em, vd: (jnp.minimum(i, NT_E - 1), 0)),
            pl.BlockSpec((TILE, H),
                         lambda i, em, vd: (jnp.maximum(i - NT_E, 0), 0)),
            pl.BlockSpec((1, I, H), lambda i, em, vd: (em[i], 0, 0)),
            pl.BlockSpec((1, H, I), lambda i, em, vd: (em[i], 0, 0)),
            pl.BlockSpec((1, 1, TILE), lambda i, em, vd: (i, 0, 0)),
        ],
        out_specs=pl.BlockSpec((TILE, H), lambda i, em, vd: (i, 0)),
    )
    kern = pl.pallas_call(
        _mlp_kernel,
        grid_spec=grid_spec,
        out_shape=jax.ShapeDtypeStruct((PTOT, H), jnp.float32),
        interpret=_INTERPRET,
    )
    return kern(em, valid, xg, x_flat, W1all, W2all,
                wslot.reshape(NT, 1, TILE))


_CCH = 8
_CNC = NT_TOK // NW // _CCH


def _combine_kernel(o_hbm, pos1_hbm, pos2_hbm, y_hbm,
                    p1_v, p2_v, g1a, g1b, g2a, g2b, ya, yb,
                    s1a, s1b, s2a, s2b, s3a, s3b, wsa, wsb):
    wid = lax.axis_index("s") * NC + lax.axis_index("c")
    tok_per_w = NT_TOK // NW
    g1 = (g1a, g1b)
    g2 = (g2a, g2b)
    yv = (ya, yb)
    s1 = (s1a, s1b)
    s2 = (s2a, s2b)
    s3 = (s3a, s3b)
    ws = (wsa, wsb)

    pltpu.sync_copy(pos1_hbm.at[pl.ds(wid * _CNC, _CNC)], p1_v)
    pltpu.sync_copy(pos2_hbm.at[pl.ds(wid * _CNC, _CNC)], p2_v)

    def start(c):
        b = c & 1
        tbase = wid * tok_per_w + c * _CCH
        return (
            pltpu.async_copy(o_hbm.at[p1_v.at[c]], g1[b], s1[b]),
            pltpu.async_copy(o_hbm.at[p2_v.at[c]], g2[b], s2[b]),
            pltpu.async_copy(o_hbm.at[pl.ds(PE + tbase, _CCH)], yv[b],
                             s3[b]),
        )

    gd = start(0)
    wd = [None, None]
    for c in range(_CNC):
        b = c & 1
        nb = 1 - b
        for d in gd:
            d.wait()
        if c + 1 < _CNC:
            if wd[nb] is not None:
                wd[nb].wait()
            gd = start(c + 1)

        def add_body(j, _):
            for r in range(_CCH):
                sl = (r, pl.ds(j * L, L))
                plsc.addupdate(yv[b].at[sl], g1[b][sl] + g2[b][sl])
            return 0
        lax.fori_loop(0, H // L, add_body, 0)

        tbase = wid * tok_per_w + c * _CCH
        wd[b] = pltpu.async_copy(yv[b], y_hbm.at[pl.ds(tbase, _CCH)], ws[b])
    for b in range(2):
        if wd[b] is not None:
            wd[b].wait()


def _combine(o, pos1, pos2):
    kern = functools.partial(
        pl.kernel,
        out_type=jax.ShapeDtypeStruct((NT_TOK, H), jnp.float32),
        mesh=plsc.VectorSubcoreMesh(**_SC_MESH),
        compiler_params=pltpu.CompilerParams(needs_layout_passes=False),
        scratch_types=[
            pltpu.VMEM((_CNC, _CCH), jnp.int32),
            pltpu.VMEM((_CNC, _CCH), jnp.int32),
            pltpu.VMEM((_CCH, H), jnp.float32),
            pltpu.VMEM((_CCH, H), jnp.float32),
            pltpu.VMEM((_CCH, H), jnp.float32),
            pltpu.VMEM((_CCH, H), jnp.float32),
            pltpu.VMEM((_CCH, H), jnp.float32),
            pltpu.VMEM((_CCH, H), jnp.float32),
            pltpu.SemaphoreType.DMA,
            pltpu.SemaphoreType.DMA,
            pltpu.SemaphoreType.DMA,
            pltpu.SemaphoreType.DMA,
            pltpu.SemaphoreType.DMA,
            pltpu.SemaphoreType.DMA,
            pltpu.SemaphoreType.DMA,
            pltpu.SemaphoreType.DMA,
        ],
        interpret=_INTERPRET,
    )(_combine_kernel)
    return kern(o, pos1.reshape(NT_TOK // _CCH, _CCH),
                pos2.reshape(NT_TOK // _CCH, _CCH))


def kernel(x, Wg, W1, W2, Ws1, Ws2):
    x_flat = x.reshape(NT_TOK, H)
    e1, e2, v1, v2, aux = _router(x_flat, Wg)

    tok_sorted, wslot, em, valid, pos1, pos2 = _route_sort(
        e1[:, 0], e2[:, 0], v1[:, 0], v2[:, 0])

    x_pk = lax.bitcast_convert_type(
        x_flat.astype(jnp.bfloat16).reshape(NT_TOK, H // 2, 2), jnp.int32)
    xg = lax.bitcast_convert_type(
        _gather_rows(tok_sorted, x_pk), jnp.bfloat16).reshape(PE, H)

    W1all = jnp.concatenate([W1, Ws1[None]], axis=0).astype(jnp.bfloat16)
    W2all = jnp.concatenate([W2, Ws2[None]], axis=0).astype(jnp.bfloat16)
    o = _expert_mlp(em, valid, xg, x_flat, W1all, W2all, wslot)

    y_flat = _combine(o, pos1, pos2)

    y = y_flat.reshape(B, T, H).astype(x.dtype)
    return y, aux[0, 0]


# speedup vs baseline: 1.3591x; 1.3591x over previous
import functools

import jax
import jax.numpy as jnp
from jax import lax
from jax.experimental import pallas as pl
from jax.experimental.pallas import tpu as pltpu
from jax.experimental.pallas import tpu_sc as plsc

_INTERPRET = False

B, T, H = 1, 2048, 1024
I = 2048
E = 8
K = 2
NT_TOK = B * T
NP = NT_TOK * K
TILE = 256
TILE_SHIFT = 8
NT_E = NP // TILE + E
PE = NT_E * TILE
NT_S = NT_TOK // TILE
NT = NT_E + NT_S
PTOT = PE + NT_TOK
EPAD = 128

NC, NS = 2, 16
NW = NC * NS
L = 16

_SC_MESH = dict(core_axis_name="c", subcore_axis_name="s",
                num_cores=NC, num_subcores=NS)


def _router_kernel(x_ref, wg_ref, e1_ref, e2_ref, v1_ref, v2_ref, aux_ref,
                   xbf_ref):
    x = x_ref[...]
    xbf_ref[...] = x.astype(jnp.bfloat16)
    wg = wg_ref[...]
    logits = lax.dot_general(x, wg, (((1,), (1,)), ((), ())),
                             preferred_element_type=jnp.float32)
    col = lax.broadcasted_iota(jnp.int32, logits.shape, 1)
    valid = col < E
    z = jnp.where(valid, logits, -1e30)
    zmax = jnp.max(z, axis=1, keepdims=True)
    p = jnp.where(valid, jnp.exp(z - zmax), 0.0)
    probs = p / jnp.sum(p, axis=1, keepdims=True)
    v1 = jnp.max(probs, axis=1, keepdims=True)
    e1 = jnp.min(jnp.where(probs >= v1, col, EPAD), axis=1, keepdims=True)
    probs2 = jnp.where(col == e1, -1.0, probs)
    v2 = jnp.max(probs2, axis=1, keepdims=True)
    e2 = jnp.min(jnp.where(probs2 >= v2, col, EPAD), axis=1, keepdims=True)
    e1_ref[...] = e1
    e2_ref[...] = e2
    v1_ref[...] = v1
    v2_ref[...] = v2
    cnt = jnp.sum((col == e1).astype(jnp.float32)
                  + (col == e2).astype(jnp.float32), axis=0, keepdims=True)
    imp = jnp.mean(probs, axis=0, keepdims=True)
    aux_ref[...] = jnp.sum(imp * cnt, axis=1, keepdims=True) * (
        float(E) / float(NT_TOK * K))


def _router(x_flat, Wg):
    wg_pad = jnp.zeros((EPAD, H), jnp.float32).at[:E].set(Wg)
    kern = pl.pallas_call(
        _router_kernel,
        out_shape=(
            jax.ShapeDtypeStruct((T, 1), jnp.int32),
            jax.ShapeDtypeStruct((T, 1), jnp.int32),
            jax.ShapeDtypeStruct((T, 1), jnp.float32),
            jax.ShapeDtypeStruct((T, 1), jnp.float32),
            jax.ShapeDtypeStruct((1, 1), jnp.float32),
            jax.ShapeDtypeStruct((T, H), jnp.bfloat16),
        ),
        interpret=_INTERPRET,
    )
    return kern(x_flat, wg_pad)


def _route_kernel(e1_hbm, e2_hbm, v1_hbm, v2_hbm,
                  tok_out, w_out, em_out, valid_out, pos1_out, pos2_out,
                  e1_v, e2_v, v1_v, v2_v, tok_v, w_v, em_v, valid_v,
                  pos1_v, pos2_v):
    cid = lax.axis_index("c")
    sid = lax.axis_index("s")

    @pl.when(jnp.logical_and(cid == 0, sid == 0))
    def _():
        pltpu.sync_copy(e1_hbm, e1_v)
        pltpu.sync_copy(e2_hbm, e2_v)
        pltpu.sync_copy(v1_hbm, v1_v)
        pltpu.sync_copy(v2_hbm, v2_v)

        zero_i = jnp.zeros((L,), jnp.int32)
        zero_f = jnp.zeros((L,), jnp.float32)
        one_f = jnp.ones((L,), jnp.float32)
        lane = lax.iota(jnp.int32, L)

        def zero_body(i, _):
            tok_v[pl.ds(i * L, L)] = zero_i
            w_v[pl.ds(i * L, L)] = zero_f
            return 0
        lax.fori_loop(0, PE // L, zero_body, 0, unroll=4)

        def shared_body(i, _):
            w_v[pl.ds(PE + i * L, L)] = one_f
            return 0
        lax.fori_loop(0, NT_TOK // L, shared_body, 0, unroll=4)

        def cnt_body(i, c):
            a = e1_v[pl.ds(i * L, L)]
            b = e2_v[pl.ds(i * L, L)]
            return tuple(
                c[e]
                + jnp.sum(jnp.where(a == e, 1, 0))
                + jnp.sum(jnp.where(b == e, 1, 0))
                for e in range(E))
        counts = lax.fori_loop(0, NT_TOK // L, cnt_body,
                               (jnp.int32(0),) * E)

        tiles = [lax.shift_right_logical(counts[e] + (TILE - 1), TILE_SHIFT)
                 for e in range(E)]
        bases = [jnp.int32(0)]
        for e in range(E - 1):
            bases.append(bases[e] + lax.shift_left(tiles[e], TILE_SHIFT))
        cum_tiles = [tiles[0]]
        for e in range(1, E):
            cum_tiles.append(cum_tiles[e - 1] + tiles[e])
        used = cum_tiles[E - 1]

        for j in range(NT // L):
            ti = j * L + lane
            em = zero_i
            for e in range(E):
                em = em + jnp.where(ti >= cum_tiles[e], 1, 0)
            is_shared = ti >= NT_E
            em_v[pl.ds(j * L, L)] = jnp.where(is_shared, E, em)
            vd = jnp.logical_or(ti < used, is_shared)
            valid_v[pl.ds(j * L, L)] = jnp.where(vd, 1, 0)

        def make_pos_body(e_ref, v_ref, p_ref):
            def body(i, b):
                ids = e_ref[pl.ds(i * L, L)]
                wv = v_ref[pl.ds(i * L, L)]
                pos = zero_i
                nb = []
                for e in range(E):
                    m = ids == e
                    ones = jnp.where(m, 1, 0)
                    pf = plsc.cumsum(ones)
                    pos = jnp.where(m, b[e] + pf - 1, pos)
                    nb.append(b[e] + jnp.sum(ones))
                tok = i * L + lane
                plsc.store_scatter(tok_v, [pos], tok)
                plsc.store_scatter(w_v, [pos], wv)
                p_ref[pl.ds(i * L, L)] = pos
                return tuple(nb)
            return body

        bases = lax.fori_loop(0, NT_TOK // L,
                              make_pos_body(e1_v, v1_v, pos1_v),
                              tuple(bases))
        lax.fori_loop(0, NT_TOK // L,
                      make_pos_body(e2_v, v2_v, pos2_v), bases)

        pltpu.sync_copy(tok_v, tok_out)
        pltpu.sync_copy(w_v, w_out)
        pltpu.sync_copy(em_v, em_out)
        pltpu.sync_copy(valid_v, valid_out)
        pltpu.sync_copy(pos1_v, pos1_out)
        pltpu.sync_copy(pos2_v, pos2_out)


def _route_sort(e1, e2, v1, v2):
    kern = functools.partial(
        pl.kernel,
        out_type=(
            jax.ShapeDtypeStruct((PE,), jnp.int32),
            jax.ShapeDtypeStruct((PTOT,), jnp.float32),
            jax.ShapeDtypeStruct((NT,), jnp.int32),
            jax.ShapeDtypeStruct((NT,), jnp.int32),
            jax.ShapeDtypeStruct((NT_TOK,), jnp.int32),
            jax.ShapeDtypeStruct((NT_TOK,), jnp.int32),
        ),
        mesh=plsc.VectorSubcoreMesh(**_SC_MESH),
        compiler_params=pltpu.CompilerParams(needs_layout_passes=False),
        scratch_types=[
            pltpu.VMEM((NT_TOK,), jnp.int32),
            pltpu.VMEM((NT_TOK,), jnp.int32),
            pltpu.VMEM((NT_TOK,), jnp.float32),
            pltpu.VMEM((NT_TOK,), jnp.float32),
            pltpu.VMEM((PE,), jnp.int32),
            pltpu.VMEM((PTOT,), jnp.float32),
            pltpu.VMEM((NT,), jnp.int32),
            pltpu.VMEM((NT,), jnp.int32),
            pltpu.VMEM((NT_TOK,), jnp.int32),
            pltpu.VMEM((NT_TOK,), jnp.int32),
        ],
        interpret=_INTERPRET,
    )(_route_kernel)
    return kern(e1, e2, v1, v2)


NJ = 2
IB = I // NJ


def _mlp_kernel(em_ref, valid_ref, tok_ref, xall_ref, xs_ref,
                w1_ref, w2_ref, ws1_ref, ws2_ref, wv_ref, out_ref, xt_ref):
    i = pl.program_id(0)
    j = pl.program_id(1)
    is_expert = i < NT_E

    @pl.when(jnp.logical_and(valid_ref[i] != 0, is_expert))
    def _expert():
        @pl.when(j == 0)
        def _gather():
            tok = tok_ref[0, 0]
            cols = lax.broadcasted_iota(jnp.int32, (TILE, NT_TOK), 1)
            oh = (cols == tok[:, None]).astype(jnp.bfloat16)
            xt_ref[...] = lax.dot_general(
                oh, xall_ref[...], (((1,), (0,)), ((), ())),
                preferred_element_type=jnp.float32).astype(jnp.bfloat16)

        xt = xt_ref[...]
        w1 = w1_ref[0].astype(jnp.bfloat16)
        h = lax.dot_general(xt, w1, (((1,), (1,)), ((), ())),
                            preferred_element_type=jnp.float32)
        a = (h * jax.nn.sigmoid(h)).astype(jnp.bfloat16)
        w2 = w2_ref[0].astype(jnp.bfloat16)
        o = lax.dot_general(a, w2, (((1,), (1,)), ((), ())),
                            preferred_element_type=jnp.float32)
        o = o * wv_ref[0, 0][:, None]

        @pl.when(j == 0)
        def _():
            out_ref[...] = o

        @pl.when(j != 0)
        def _():
            out_ref[...] += o

    @pl.when(jnp.logical_not(is_expert))
    def _shared():
        xt = xs_ref[...]
        w1 = ws1_ref[...].astype(jnp.bfloat16)
        h = lax.dot_general(xt, w1, (((1,), (1,)), ((), ())),
                            preferred_element_type=jnp.float32)
        a = (h * jax.nn.sigmoid(h)).astype(jnp.bfloat16)
        w2 = ws2_ref[...].astype(jnp.bfloat16)
        o = lax.dot_general(a, w2, (((1,), (1,)), ((), ())),
                            preferred_element_type=jnp.float32)

        @pl.when(j == 0)
        def _():
            out_ref[...] = o

        @pl.when(j != 0)
        def _():
            out_ref[...] += o


def _expert_mlp(em, valid, tok_sorted, xbf, W1, W2, Ws1, Ws2, wslot):
    grid_spec = pltpu.PrefetchScalarGridSpec(
        num_scalar_prefetch=2,
        grid=(NT, NJ),
        in_specs=[
            pl.BlockSpec((1, 1, TILE),
                         lambda i, j, em, vd: (jnp.minimum(i, NT_E - 1),
                                               0, 0)),
            pl.BlockSpec((NT_TOK, H), lambda i, j, em, vd: (0, 0)),
            pl.BlockSpec((TILE, H),
                         lambda i, j, em, vd: (jnp.maximum(i - NT_E, 0), 0)),
            pl.BlockSpec((1, IB, H),
                         lambda i, j, em, vd: (jnp.minimum(em[i], E - 1),
                                               j, 0)),
            pl.BlockSpec((1, H, IB),
                         lambda i, j, em, vd: (jnp.minimum(em[i], E - 1),
                                               0, j)),
            pl.BlockSpec((IB, H), lambda i, j, em, vd: (j, 0)),
            pl.BlockSpec((H, IB), lambda i, j, em, vd: (0, j)),
            pl.BlockSpec((1, 1, TILE), lambda i, j, em, vd: (i, 0, 0)),
        ],
        out_specs=pl.BlockSpec((TILE, H), lambda i, j, em, vd: (i, 0)),
        scratch_shapes=[pltpu.VMEM((TILE, H), jnp.bfloat16)],
    )
    kern = pl.pallas_call(
        _mlp_kernel,
        grid_spec=grid_spec,
        out_shape=jax.ShapeDtypeStruct((PTOT, H), jnp.float32),
        interpret=_INTERPRET,
    )
    return kern(em, valid, tok_sorted.reshape(NT_E, 1, TILE), xbf, xbf,
                W1, W2, Ws1, Ws2, wslot.reshape(NT, 1, TILE))


_CCH = 8
_CNC = NT_TOK // NW // _CCH


def _combine_kernel(o_hbm, pos1_hbm, pos2_hbm, y_hbm,
                    p1_v, p2_v, g1a, g1b, g2a, g2b, ya, yb,
                    s1a, s1b, s2a, s2b, s3a, s3b, wsa, wsb):
    wid = lax.axis_index("s") * NC + lax.axis_index("c")
    tok_per_w = NT_TOK // NW
    g1 = (g1a, g1b)
    g2 = (g2a, g2b)
    yv = (ya, yb)
    s1 = (s1a, s1b)
    s2 = (s2a, s2b)
    s3 = (s3a, s3b)
    ws = (wsa, wsb)

    pltpu.sync_copy(pos1_hbm.at[pl.ds(wid * _CNC, _CNC)], p1_v)
    pltpu.sync_copy(pos2_hbm.at[pl.ds(wid * _CNC, _CNC)], p2_v)

    def start(c):
        b = c & 1
        tbase = wid * tok_per_w + c * _CCH
        return (
            pltpu.async_copy(o_hbm.at[p1_v.at[c]], g1[b], s1[b]),
            pltpu.async_copy(o_hbm.at[p2_v.at[c]], g2[b], s2[b]),
            pltpu.async_copy(o_hbm.at[pl.ds(PE + tbase, _CCH)], yv[b],
                             s3[b]),
        )

    gd = start(0)
    wd = [None, None]
    for c in range(_CNC):
        b = c & 1
        nb = 1 - b
        for d in gd:
            d.wait()
        if c + 1 < _CNC:
            if wd[nb] is not None:
                wd[nb].wait()
            gd = start(c + 1)

        def add_body(j, _):
            for r in range(_CCH):
                sl = (r, pl.ds(j * L, L))
                plsc.addupdate(yv[b].at[sl], g1[b][sl] + g2[b][sl])
            return 0
        lax.fori_loop(0, H // L, add_body, 0)

        tbase = wid * tok_per_w + c * _CCH
        wd[b] = pltpu.async_copy(yv[b], y_hbm.at[pl.ds(tbase, _CCH)], ws[b])
    for b in range(2):
        if wd[b] is not None:
            wd[b].wait()


def _combine(o, pos1, pos2):
    kern = functools.partial(
        pl.kernel,
        out_type=jax.ShapeDtypeStruct((NT_TOK, H), jnp.float32),
        mesh=plsc.VectorSubcoreMesh(**_SC_MESH),
        compiler_params=pltpu.CompilerParams(needs_layout_passes=False),
        scratch_types=[
            pltpu.VMEM((_CNC, _CCH), jnp.int32),
            pltpu.VMEM((_CNC, _CCH), jnp.int32),
            pltpu.VMEM((_CCH, H), jnp.float32),
            pltpu.VMEM((_CCH, H), jnp.float32),
            pltpu.VMEM((_CCH, H), jnp.float32),
            pltpu.VMEM((_CCH, H), jnp.float32),
            pltpu.VMEM((_CCH, H), jnp.float32),
            pltpu.VMEM((_CCH, H), jnp.float32),
            pltpu.SemaphoreType.DMA,
            pltpu.SemaphoreType.DMA,
            pltpu.SemaphoreType.DMA,
            pltpu.SemaphoreType.DMA,
            pltpu.SemaphoreType.DMA,
            pltpu.SemaphoreType.DMA,
            pltpu.SemaphoreType.DMA,
            pltpu.SemaphoreType.DMA,
        ],
        interpret=_INTERPRET,
    )(_combine_kernel)
    return kern(o, pos1.reshape(NT_TOK // _CCH, _CCH),
                pos2.reshape(NT_TOK // _CCH, _CCH))


def kernel(x, Wg, W1, W2, Ws1, Ws2):
    x_flat = x.reshape(NT_TOK, H)
    e1, e2, v1, v2, aux, xbf = _router(x_flat, Wg)

    tok_sorted, wslot, em, valid, pos1, pos2 = _route_sort(
        e1[:, 0], e2[:, 0], v1[:, 0], v2[:, 0])

    o = _expert_mlp(em, valid, tok_sorted, xbf, W1, W2, Ws1, Ws2, wslot)

    y_flat = _combine(o, pos1, pos2)

    y = y_flat.reshape(B, T, H).astype(x.dtype)
    return y, aux[0, 0]


# speedup vs baseline: 2.1956x; 1.6154x over previous
import functools

import jax
import jax.numpy as jnp
from jax import lax
from jax.experimental import pallas as pl
from jax.experimental.pallas import tpu as pltpu
from jax.experimental.pallas import tpu_sc as plsc

_INTERPRET = False

B, T, H = 1, 2048, 1024
I = 2048
E = 8
K = 2
NT_TOK = B * T
NP = NT_TOK * K
TILE = 128
TILE_SHIFT = 7
NT_E = NP // TILE + E
NTE_PAD = 48
PE = NT_E * TILE
SHTILE = 256
NT_S = NT_TOK // SHTILE
EPAD = 128

NC, NS = 2, 16
NW = NC * NS
L = 16

_SC_MESH = dict(core_axis_name="c", subcore_axis_name="s",
                num_cores=NC, num_subcores=NS)


def _router_kernel(x_ref, wg_ref, e1_ref, e2_ref, v1_ref, v2_ref, aux_ref,
                   xbf_ref):
    x = x_ref[...]
    xbf_ref[...] = x.astype(jnp.bfloat16)
    wg = wg_ref[...]
    logits = lax.dot_general(x, wg, (((1,), (1,)), ((), ())),
                             preferred_element_type=jnp.float32)
    col = lax.broadcasted_iota(jnp.int32, logits.shape, 1)
    valid = col < E
    z = jnp.where(valid, logits, -1e30)
    zmax = jnp.max(z, axis=1, keepdims=True)
    p = jnp.where(valid, jnp.exp(z - zmax), 0.0)
    probs = p / jnp.sum(p, axis=1, keepdims=True)
    v1 = jnp.max(probs, axis=1, keepdims=True)
    e1 = jnp.min(jnp.where(probs >= v1, col, EPAD), axis=1, keepdims=True)
    probs2 = jnp.where(col == e1, -1.0, probs)
    v2 = jnp.max(probs2, axis=1, keepdims=True)
    e2 = jnp.min(jnp.where(probs2 >= v2, col, EPAD), axis=1, keepdims=True)
    e1_ref[...] = e1
    e2_ref[...] = e2
    v1_ref[...] = v1
    v2_ref[...] = v2
    cnt = jnp.sum((col == e1).astype(jnp.float32)
                  + (col == e2).astype(jnp.float32), axis=0, keepdims=True)
    imp = jnp.mean(probs, axis=0, keepdims=True)
    aux_ref[...] = jnp.sum(imp * cnt, axis=1, keepdims=True) * (
        float(E) / float(NT_TOK * K))


def _router(x_flat, Wg):
    wg_pad = jnp.zeros((EPAD, H), jnp.float32).at[:E].set(Wg)
    kern = pl.pallas_call(
        _router_kernel,
        out_shape=(
            jax.ShapeDtypeStruct((T, 1), jnp.int32),
            jax.ShapeDtypeStruct((T, 1), jnp.int32),
            jax.ShapeDtypeStruct((T, 1), jnp.float32),
            jax.ShapeDtypeStruct((T, 1), jnp.float32),
            jax.ShapeDtypeStruct((1, 1), jnp.float32),
            jax.ShapeDtypeStruct((T, H), jnp.bfloat16),
        ),
        interpret=_INTERPRET,
    )
    return kern(x_flat, wg_pad)


def _route_kernel(e1_hbm, e2_hbm, v1_hbm, v2_hbm,
                  tok_out, w_out, em_out, valid_out, pos1_out, pos2_out,
                  e1_v, e2_v, v1_v, v2_v, tok_v, w_v, em_v, valid_v,
                  pos1_v, pos2_v):
    cid = lax.axis_index("c")
    sid = lax.axis_index("s")

    @pl.when(jnp.logical_and(cid == 0, sid == 0))
    def _():
        pltpu.sync_copy(e1_hbm, e1_v)
        pltpu.sync_copy(e2_hbm, e2_v)
        pltpu.sync_copy(v1_hbm, v1_v)
        pltpu.sync_copy(v2_hbm, v2_v)

        zero_i = jnp.zeros((L,), jnp.int32)
        zero_f = jnp.zeros((L,), jnp.float32)
        one_f = jnp.ones((L,), jnp.float32)
        lane = lax.iota(jnp.int32, L)

        def zero_body(i, _):
            tok_v[pl.ds(i * L, L)] = zero_i
            w_v[pl.ds(i * L, L)] = zero_f
            return 0
        lax.fori_loop(0, PE // L, zero_body, 0, unroll=4)
        del one_f

        def cnt_body(i, c):
            a = e1_v[pl.ds(i * L, L)]
            b = e2_v[pl.ds(i * L, L)]
            return tuple(
                c[e]
                + jnp.sum(jnp.where(a == e, 1, 0))
                + jnp.sum(jnp.where(b == e, 1, 0))
                for e in range(E))
        counts = lax.fori_loop(0, NT_TOK // L, cnt_body,
                               (jnp.int32(0),) * E)

        tiles = [lax.shift_right_logical(counts[e] + (TILE - 1), TILE_SHIFT)
                 for e in range(E)]
        bases = [jnp.int32(0)]
        for e in range(E - 1):
            bases.append(bases[e] + lax.shift_left(tiles[e], TILE_SHIFT))
        cum_tiles = [tiles[0]]
        for e in range(1, E):
            cum_tiles.append(cum_tiles[e - 1] + tiles[e])
        used = cum_tiles[E - 1]

        for j in range(NTE_PAD // L):
            ti = j * L + lane
            em = zero_i
            for e in range(E):
                em = em + jnp.where(ti >= cum_tiles[e], 1, 0)
            is_shared = ti >= NT_E
            em_v[pl.ds(j * L, L)] = jnp.where(is_shared, E, em)
            vd = jnp.logical_or(ti < used, is_shared)
            valid_v[pl.ds(j * L, L)] = jnp.where(vd, 1, 0)

        def make_pos_body(e_ref, v_ref, p_ref):
            def body(i, b):
                ids = e_ref[pl.ds(i * L, L)]
                wv = v_ref[pl.ds(i * L, L)]
                pos = zero_i
                nb = []
                for e in range(E):
                    m = ids == e
                    ones = jnp.where(m, 1, 0)
                    pf = plsc.cumsum(ones)
                    pos = jnp.where(m, b[e] + pf - 1, pos)
                    nb.append(b[e] + jnp.sum(ones))
                tok = i * L + lane
                plsc.store_scatter(tok_v, [pos], tok)
                plsc.store_scatter(w_v, [pos], wv)
                p_ref[pl.ds(i * L, L)] = pos
                return tuple(nb)
            return body

        bases = lax.fori_loop(0, NT_TOK // L,
                              make_pos_body(e1_v, v1_v, pos1_v),
                              tuple(bases))
        lax.fori_loop(0, NT_TOK // L,
                      make_pos_body(e2_v, v2_v, pos2_v), bases)

        pltpu.sync_copy(tok_v, tok_out)
        pltpu.sync_copy(w_v, w_out)
        pltpu.sync_copy(em_v, em_out)
        pltpu.sync_copy(valid_v, valid_out)
        pltpu.sync_copy(pos1_v, pos1_out)
        pltpu.sync_copy(pos2_v, pos2_out)


def _route_sort(e1, e2, v1, v2):
    kern = functools.partial(
        pl.kernel,
        out_type=(
            jax.ShapeDtypeStruct((PE,), jnp.int32),
            jax.ShapeDtypeStruct((PE,), jnp.float32),
            jax.ShapeDtypeStruct((NTE_PAD,), jnp.int32),
            jax.ShapeDtypeStruct((NTE_PAD,), jnp.int32),
            jax.ShapeDtypeStruct((NT_TOK,), jnp.int32),
            jax.ShapeDtypeStruct((NT_TOK,), jnp.int32),
        ),
        mesh=plsc.VectorSubcoreMesh(**_SC_MESH),
        compiler_params=pltpu.CompilerParams(needs_layout_passes=False),
        scratch_types=[
            pltpu.VMEM((NT_TOK,), jnp.int32),
            pltpu.VMEM((NT_TOK,), jnp.int32),
            pltpu.VMEM((NT_TOK,), jnp.float32),
            pltpu.VMEM((NT_TOK,), jnp.float32),
            pltpu.VMEM((PE,), jnp.int32),
            pltpu.VMEM((PE,), jnp.float32),
            pltpu.VMEM((NTE_PAD,), jnp.int32),
            pltpu.VMEM((NTE_PAD,), jnp.int32),
            pltpu.VMEM((NT_TOK,), jnp.int32),
            pltpu.VMEM((NT_TOK,), jnp.int32),
        ],
        interpret=_INTERPRET,
    )(_route_kernel)
    return kern(e1, e2, v1, v2)


def _mlp_kernel(em_ref, valid_ref, tok_ref, xall_ref,
                w1_ref, w2_ref, wv_ref, out_ref, xt_ref, w1b_ref, w2b_ref):
    i = pl.program_id(0)

    @pl.when(valid_ref[i] != 0)
    def _():
        prev_em = jnp.where(i > 0, em_ref[jnp.maximum(i - 1, 0)], -1)

        @pl.when(em_ref[i] != prev_em)
        def _cast():
            w1b_ref[...] = w1_ref[0].astype(jnp.bfloat16)
            w2b_ref[...] = w2_ref[0].astype(jnp.bfloat16)

        tok = tok_ref[0, 0]
        cols = lax.broadcasted_iota(jnp.int32, (TILE, NT_TOK), 1)
        oh = (cols == tok[:, None]).astype(jnp.bfloat16)
        xt_ref[...] = lax.dot_general(
            oh, xall_ref[...], (((1,), (0,)), ((), ())),
            preferred_element_type=jnp.float32).astype(jnp.bfloat16)

        xt = xt_ref[...]
        h = lax.dot_general(xt, w1b_ref[...], (((1,), (1,)), ((), ())),
                            preferred_element_type=jnp.float32)
        a = (h * jax.nn.sigmoid(h)).astype(jnp.bfloat16)
        o = lax.dot_general(a, w2b_ref[...], (((1,), (1,)), ((), ())),
                            preferred_element_type=jnp.float32)
        out_ref[...] = o * wv_ref[0, 0][:, None]


def _expert_mlp(em, valid, tok_sorted, xbf, W1, W2, wslot):
    grid_spec = pltpu.PrefetchScalarGridSpec(
        num_scalar_prefetch=2,
        grid=(NT_E,),
        in_specs=[
            pl.BlockSpec((1, 1, TILE), lambda i, em, vd: (i, 0, 0)),
            pl.BlockSpec((NT_TOK, H), lambda i, em, vd: (0, 0)),
            pl.BlockSpec((1, I, H),
                         lambda i, em, vd: (jnp.minimum(em[i], E - 1), 0, 0)),
            pl.BlockSpec((1, H, I),
                         lambda i, em, vd: (jnp.minimum(em[i], E - 1), 0, 0)),
            pl.BlockSpec((1, 1, TILE), lambda i, em, vd: (i, 0, 0)),
        ],
        out_specs=pl.BlockSpec((TILE, H), lambda i, em, vd: (i, 0)),
        scratch_shapes=[
            pltpu.VMEM((TILE, H), jnp.bfloat16),
            pltpu.VMEM((I, H), jnp.bfloat16),
            pltpu.VMEM((H, I), jnp.bfloat16),
        ],
    )
    kern = pl.pallas_call(
        _mlp_kernel,
        grid_spec=grid_spec,
        out_shape=jax.ShapeDtypeStruct((PE, H), jnp.float32),
        interpret=_INTERPRET,
    )
    return kern(em, valid, tok_sorted.reshape(NT_E, 1, TILE), xbf,
                W1, W2, wslot.reshape(NT_E, 1, TILE))


def _shared_kernel(xs_ref, ws1_ref, ws2_ref, out_ref, w1b_ref, w2b_ref):
    i = pl.program_id(0)

    @pl.when(i == 0)
    def _cast():
        w1b_ref[...] = ws1_ref[...].astype(jnp.bfloat16)
        w2b_ref[...] = ws2_ref[...].astype(jnp.bfloat16)

    xt = xs_ref[...]
    h = lax.dot_general(xt, w1b_ref[...], (((1,), (1,)), ((), ())),
                        preferred_element_type=jnp.float32)
    a = (h * jax.nn.sigmoid(h)).astype(jnp.bfloat16)
    out_ref[...] = lax.dot_general(a, w2b_ref[...], (((1,), (1,)), ((), ())),
                                   preferred_element_type=jnp.float32)


def _shared_mlp(xbf, Ws1, Ws2):
    kern = pl.pallas_call(
        _shared_kernel,
        grid=(NT_S,),
        in_specs=[
            pl.BlockSpec((SHTILE, H), lambda i: (i, 0)),
            pl.BlockSpec((I, H), lambda i: (0, 0)),
            pl.BlockSpec((H, I), lambda i: (0, 0)),
        ],
        out_specs=pl.BlockSpec((SHTILE, H), lambda i: (i, 0)),
        scratch_shapes=[
            pltpu.VMEM((I, H), jnp.bfloat16),
            pltpu.VMEM((H, I), jnp.bfloat16),
        ],
        out_shape=jax.ShapeDtypeStruct((NT_TOK, H), jnp.float32),
        interpret=_INTERPRET,
    )
    return kern(xbf, Ws1, Ws2)


_CCH = 8
_CNC = NT_TOK // NW // _CCH


def _combine_kernel(o_hbm, osh_hbm, pos1_hbm, pos2_hbm, y_hbm,
                    p1_v, p2_v, g1a, g1b, g2a, g2b, ya, yb,
                    s1a, s1b, s2a, s2b, s3a, s3b, wsa, wsb):
    wid = lax.axis_index("s") * NC + lax.axis_index("c")
    tok_per_w = NT_TOK // NW
    g1 = (g1a, g1b)
    g2 = (g2a, g2b)
    yv = (ya, yb)
    s1 = (s1a, s1b)
    s2 = (s2a, s2b)
    s3 = (s3a, s3b)
    ws = (wsa, wsb)

    pltpu.sync_copy(pos1_hbm.at[pl.ds(wid * _CNC, _CNC)], p1_v)
    pltpu.sync_copy(pos2_hbm.at[pl.ds(wid * _CNC, _CNC)], p2_v)

    def start(c):
        b = c & 1
        tbase = wid * tok_per_w + c * _CCH
        return (
            pltpu.async_copy(o_hbm.at[p1_v.at[c]], g1[b], s1[b]),
            pltpu.async_copy(o_hbm.at[p2_v.at[c]], g2[b], s2[b]),
            pltpu.async_copy(osh_hbm.at[pl.ds(tbase, _CCH)], yv[b],
                             s3[b]),
        )

    gd = start(0)
    wd = [None, None]
    for c in range(_CNC):
        b = c & 1
        nb = 1 - b
        for d in gd:
            d.wait()
        if c + 1 < _CNC:
            if wd[nb] is not None:
                wd[nb].wait()
            gd = start(c + 1)

        def add_body(j, _):
            for r in range(_CCH):
                sl = (r, pl.ds(j * L, L))
                plsc.addupdate(yv[b].at[sl], g1[b][sl] + g2[b][sl])
            return 0
        lax.fori_loop(0, H // L, add_body, 0)

        tbase = wid * tok_per_w + c * _CCH
        wd[b] = pltpu.async_copy(yv[b], y_hbm.at[pl.ds(tbase, _CCH)], ws[b])
    for b in range(2):
        if wd[b] is not None:
            wd[b].wait()


def _combine(o, osh, pos1, pos2):
    kern = functools.partial(
        pl.kernel,
        out_type=jax.ShapeDtypeStruct((NT_TOK, H), jnp.float32),
        mesh=plsc.VectorSubcoreMesh(**_SC_MESH),
        compiler_params=pltpu.CompilerParams(needs_layout_passes=False),
        scratch_types=[
            pltpu.VMEM((_CNC, _CCH), jnp.int32),
            pltpu.VMEM((_CNC, _CCH), jnp.int32),
            pltpu.VMEM((_CCH, H), jnp.float32),
            pltpu.VMEM((_CCH, H), jnp.float32),
            pltpu.VMEM((_CCH, H), jnp.float32),
            pltpu.VMEM((_CCH, H), jnp.float32),
            pltpu.VMEM((_CCH, H), jnp.float32),
            pltpu.VMEM((_CCH, H), jnp.float32),
            pltpu.SemaphoreType.DMA,
            pltpu.SemaphoreType.DMA,
            pltpu.SemaphoreType.DMA,
            pltpu.SemaphoreType.DMA,
            pltpu.SemaphoreType.DMA,
            pltpu.SemaphoreType.DMA,
            pltpu.SemaphoreType.DMA,
            pltpu.SemaphoreType.DMA,
        ],
        interpret=_INTERPRET,
    )(_combine_kernel)
    return kern(o, osh, pos1.reshape(NT_TOK // _CCH, _CCH),
                pos2.reshape(NT_TOK // _CCH, _CCH))


def kernel(x, Wg, W1, W2, Ws1, Ws2):
    x_flat = x.reshape(NT_TOK, H)
    e1, e2, v1, v2, aux, xbf = _router(x_flat, Wg)

    tok_sorted, wslot, em, valid, pos1, pos2 = _route_sort(
        e1[:, 0], e2[:, 0], v1[:, 0], v2[:, 0])

    o = _expert_mlp(em, valid, tok_sorted, xbf, W1, W2, wslot)
    osh = _shared_mlp(xbf, Ws1, Ws2)

    y_flat = _combine(o, osh, pos1, pos2)

    y = y_flat.reshape(B, T, H).astype(x.dtype)
    return y, aux[0, 0]
